# baseline (speedup 1.0000x reference)
import functools
import jax
import jax.numpy as jnp
from jax import lax
from jax.experimental import pallas as pl
from jax.experimental.pallas import tpu as pltpu, tpu_sc as plsc

_L = 16
_NBKT = 256


def _make_sc_kernel(b, n, k, nc=2, ns=16, interpret=False):
    nw = nc * ns
    rpw = b // nw
    nv = n // _L
    mesh = plsc.VectorSubcoreMesh(
        core_axis_name="c", subcore_axis_name="s",
        num_cores=nc, num_subcores=ns)

    def body(x_hbm, out_hbm, row_v, cand_v, hist_v, tot_v):
        cid = lax.axis_index("c")
        sid = lax.axis_index("s")
        wid = sid * nc + cid
        lanes = lax.broadcasted_iota(jnp.int32, (_L,), 0)
        laneoff = lanes * _NBKT
        zeros16 = jnp.zeros((_L,), jnp.int32)
        ones16 = jnp.ones((_L,), jnp.int32)
        kk = jnp.int32(k)

        def monotone_key(v):
            xi = plsc.bitcast(v, jnp.int32)
            return xi ^ ((xi >> 31) & jnp.int32(0x7FFFFFFF))

        for i in range(rpw):
            row = wid * rpw + i
            with jax.named_scope("ph_dma_in"):
                pltpu.sync_copy(x_hbm.at[row], row_v)

            sc_hist = jax.named_scope("ph_hist"); sc_hist.__enter__()
            @plsc.parallel_loop(0, _NBKT, unroll=8)
            def _(j):
                hist_v[pl.ds(j * _L, _L)] = zeros16

            @plsc.parallel_loop(0, nv, unroll=8)
            def _(j):
                key = monotone_key(row_v[pl.ds(j * _L, _L)])
                bkt = (key >> 24) + 128
                plsc.addupdate_scatter(hist_v, [laneoff + bkt], ones16)

            def t_body(g, _):
                acc = zeros16
                for l in range(_L):
                    acc = acc + hist_v[pl.ds(l * _NBKT + g * _L, _L)]
                tot_v[pl.ds(g * _L, _L)] = acc
                return 0
            lax.fori_loop(0, _NBKT // _L, t_body, 0)

            sc_hist.__exit__(None, None, None)
            sc_red = jax.named_scope("ph_reduce"); sc_red.__enter__()
            def nb_body(t, carry):
                hi, cntb = carry
                g = jnp.int32(_NBKT // _L - 1) - t
                tg = tot_v[pl.ds(g * _L, _L)]
                suf = lax.rev(plsc.cumsum(lax.rev(tg, (0,))), (0,)) + hi
                cntb = cntb + jnp.sum((suf >= kk).astype(jnp.int32))
                return (hi + jnp.sum(tg), cntb)
            _, nbcnt = lax.fori_loop(0, _NBKT // _L, nb_body,
                                     (jnp.int32(0), jnp.int32(0)))
            b1 = nbcnt - 1

            def ca_body(g, acc):
                tg = tot_v[pl.ds(g * _L, _L)]
                bid = g * _L + lanes
                return acc + jnp.sum(jnp.where(bid > b1, tg, 0))
            c_above = lax.fori_loop(0, _NBKT // _L, ca_body, jnp.int32(0))

            sc_red.__exit__(None, None, None)
            sc_cpt = jax.named_scope("ph_compact"); sc_cpt.__enter__()
            def c_body(j, ptr):
                key = monotone_key(row_v[pl.ds(j * _L, _L)])
                bkt = (key >> 24) + 128
                m = bkt == b1
                cs = plsc.cumsum(m.astype(jnp.int32))
                addr = ptr + cs - 1
                plsc.store_scatter(cand_v, [addr], j * _L + lanes, mask=m)
                return ptr + jnp.sum(m.astype(jnp.int32))
            ptr = plsc.parallel_loop(0, nv, carry=zeros16, unroll=4)(c_body)
            m1 = jnp.max(ptr)
            nv_c = (m1 + _L - 1) // _L

            sc_cpt.__exit__(None, None, None)
            sc_bs = jax.named_scope("ph_bsearch"); sc_bs.__enter__()
            key_top = (b1 - 128) << 24

            def gather_keys(j):
                valid = (j * _L + lanes) < m1
                idxv = cand_v[pl.ds(j * _L, _L)]
                xv = plsc.load_gather(row_v, [idxv], mask=valid)
                return monotone_key(xv), idxv, valid

            def bit_body(t, prefix):
                cand_t = prefix | (jnp.int32(1) << (jnp.int32(23) - t))

                def cnt_body(j, acc):
                    keyv, _, valid = gather_keys(j)
                    ge = (keyv >= cand_t) & valid
                    return acc + ge.astype(jnp.int32)
                accv = plsc.parallel_loop(0, nv_c, carry=zeros16,
                                          unroll=2)(cnt_body)
                cnt = jnp.sum(accv) + c_above
                return jnp.where(cnt >= kk, cand_t, prefix)
            thr = lax.fori_loop(0, 24, bit_body, key_top)

            sc_bs.__exit__(None, None, None)
            sc_tie = jax.named_scope("ph_tie"); sc_tie.__enter__()
            def gt_body(j, acc):
                keyv, _, valid = gather_keys(j)
                gt = (keyv > thr) & valid
                return acc + gt.astype(jnp.int32)
            cgt = jnp.sum(plsc.parallel_loop(0, nv_c, carry=zeros16,
                                             unroll=2)(gt_body)) + c_above
            r = kk - cgt

            def tie_body(j, carry):
                cnt, istar = carry
                keyv, idxv, valid = gather_keys(j)
                eq = (keyv == thr) & valid
                cs = plsc.cumsum(eq.astype(jnp.int32))
                hit = eq & ((cs + cnt) == r)
                istar = istar + jnp.sum(jnp.where(hit, idxv, 0))
                return (cnt + jnp.sum(eq.astype(jnp.int32)), istar)
            _, istar = lax.fori_loop(0, nv_c, tie_body,
                                     (jnp.int32(0), jnp.int32(0)))

            sc_tie.__exit__(None, None, None)
            sc_msk = jax.named_scope("ph_mask"); sc_msk.__enter__()
            @plsc.parallel_loop(0, nv, unroll=8)
            def _(j):
                sl = pl.ds(j * _L, _L)
                v = row_v[sl]
                key = monotone_key(v)
                gidx = j * _L + lanes
                keep = (key > thr) | ((key == thr) & (gidx <= istar))
                row_v[sl] = jnp.where(keep, v, jnp.float32(0.0))

            sc_msk.__exit__(None, None, None)
            with jax.named_scope("ph_dma_out"):
                pltpu.sync_copy(row_v, out_hbm.at[row])

    sck = pl.kernel(
        body,
        out_type=jax.ShapeDtypeStruct((b, n), jnp.float32),
        mesh=mesh,
        scratch_types=[
            pltpu.VMEM((n,), jnp.float32),
            pltpu.VMEM((n,), jnp.int32),
            pltpu.VMEM((_NBKT * _L,), jnp.int32),
            pltpu.VMEM((_NBKT,), jnp.int32),
        ],
        compiler_params=pltpu.CompilerParams(needs_layout_passes=False),
        interpret=interpret,
    )

    return sck


_kern = _make_sc_kernel(128, 32768, 512)


def kernel(x):
    return _kern(x)


# speedup vs baseline: 1.0950x; 1.0950x over previous
import functools
import jax
import jax.numpy as jnp
from jax import lax
from jax.experimental import pallas as pl
from jax.experimental.pallas import tpu as pltpu, tpu_sc as plsc

_L = 16
_NBKT = 256


def _make_sc_kernel(b, n, k, nc=2, ns=16, interpret=False):
    nw = nc * ns
    rpw = b // nw
    nv = n // _L
    mesh = plsc.VectorSubcoreMesh(
        core_axis_name="c", subcore_axis_name="s",
        num_cores=nc, num_subcores=ns)

    def body(x_hbm, out_hbm, row0_v, row1_v, cand_v, hist_v, tot_v,
             sem_in0, sem_in1, sem_out0, sem_out1):
        cid = lax.axis_index("c")
        sid = lax.axis_index("s")
        wid = sid * nc + cid
        lanes = lax.broadcasted_iota(jnp.int32, (_L,), 0)
        laneoff = lanes * _NBKT
        zeros16 = jnp.zeros((_L,), jnp.int32)
        ones16 = jnp.ones((_L,), jnp.int32)
        kk = jnp.int32(k)

        def monotone_key(v):
            xi = plsc.bitcast(v, jnp.int32)
            return xi ^ ((xi >> 31) & jnp.int32(0x7FFFFFFF))

        row_bufs = (row0_v, row1_v)
        sems_in = (sem_in0, sem_in1)
        sems_out = (sem_out0, sem_out1)
        descs_in = [None, None]
        descs_out = [None, None]
        base = wid * rpw
        descs_in[0] = pltpu.async_copy(x_hbm.at[base], row_bufs[0],
                                       sems_in[0])
        for i in range(rpw):
            p = i % 2
            row_v = row_bufs[p]
            if i + 1 < rpw:
                q = 1 - p
                if descs_out[q] is not None:
                    descs_out[q].wait()
                descs_in[q] = pltpu.async_copy(x_hbm.at[base + i + 1],
                                               row_bufs[q], sems_in[q])
            descs_in[p].wait()

            @plsc.parallel_loop(0, _NBKT, unroll=8)
            def _(j):
                hist_v[pl.ds(j * _L, _L)] = zeros16

            @plsc.parallel_loop(0, nv, unroll=8)
            def _(j):
                key = monotone_key(row_v[pl.ds(j * _L, _L)])
                bkt = (key >> 24) + 128
                plsc.addupdate_scatter(hist_v, [laneoff + bkt], ones16)

            def t_body(g, _):
                acc = zeros16
                for l in range(_L):
                    acc = acc + hist_v[pl.ds(l * _NBKT + g * _L, _L)]
                tot_v[pl.ds(g * _L, _L)] = acc
                return 0
            lax.fori_loop(0, _NBKT // _L, t_body, 0)

            def nb_body(t, carry):
                hi, cntb = carry
                g = jnp.int32(_NBKT // _L - 1) - t
                tg = tot_v[pl.ds(g * _L, _L)]
                suf = lax.rev(plsc.cumsum(lax.rev(tg, (0,))), (0,)) + hi
                cntb = cntb + jnp.sum((suf >= kk).astype(jnp.int32))
                return (hi + jnp.sum(tg), cntb)
            _, nbcnt = lax.fori_loop(0, _NBKT // _L, nb_body,
                                     (jnp.int32(0), jnp.int32(0)))
            b1 = nbcnt - 1

            def ca_body(g, acc):
                tg = tot_v[pl.ds(g * _L, _L)]
                bid = g * _L + lanes
                return acc + jnp.sum(jnp.where(bid > b1, tg, 0))
            c_above = lax.fori_loop(0, _NBKT // _L, ca_body, jnp.int32(0))

            def c_body(j, ptr):
                key = monotone_key(row_v[pl.ds(j * _L, _L)])
                bkt = (key >> 24) + 128
                m = bkt == b1
                cs = plsc.cumsum(m.astype(jnp.int32))
                addr = ptr + cs - 1
                plsc.store_scatter(cand_v, [addr], j * _L + lanes, mask=m)
                return ptr + jnp.sum(m.astype(jnp.int32))
            ptr = plsc.parallel_loop(0, nv, carry=zeros16, unroll=8)(c_body)
            m1 = jnp.max(ptr)
            nv_c = (m1 + _L - 1) // _L

            key_top = (b1 - 128) << 24

            def gather_keys(j):
                valid = (j * _L + lanes) < m1
                idxv = cand_v[pl.ds(j * _L, _L)]
                xv = plsc.load_gather(row_v, [idxv], mask=valid)
                return monotone_key(xv), idxv, valid

            def bit_body(t, prefix):
                cand_t = prefix | (jnp.int32(1) << (jnp.int32(23) - t))

                def cnt_body(j, acc):
                    keyv, _, valid = gather_keys(j)
                    ge = (keyv >= cand_t) & valid
                    return acc + ge.astype(jnp.int32)
                accv = plsc.parallel_loop(0, nv_c, carry=zeros16,
                                          unroll=4)(cnt_body)
                cnt = jnp.sum(accv) + c_above
                return jnp.where(cnt >= kk, cand_t, prefix)
            thr = lax.fori_loop(0, 24, bit_body, key_top)

            def gt_body(j, acc):
                keyv, _, valid = gather_keys(j)
                gt = (keyv > thr) & valid
                return acc + gt.astype(jnp.int32)
            cgt = jnp.sum(plsc.parallel_loop(0, nv_c, carry=zeros16,
                                             unroll=4)(gt_body)) + c_above
            r = kk - cgt

            def tie_body(j, carry):
                cnt, istar = carry
                keyv, idxv, valid = gather_keys(j)
                eq = (keyv == thr) & valid
                cs = plsc.cumsum(eq.astype(jnp.int32))
                hit = eq & ((cs + cnt) == r)
                istar = istar + jnp.sum(jnp.where(hit, idxv, 0))
                return (cnt + jnp.sum(eq.astype(jnp.int32)), istar)
            _, istar = lax.fori_loop(0, nv_c, tie_body,
                                     (jnp.int32(0), jnp.int32(0)))

            @plsc.parallel_loop(0, nv, unroll=8)
            def _(j):
                sl = pl.ds(j * _L, _L)
                v = row_v[sl]
                key = monotone_key(v)
                gidx = j * _L + lanes
                keep = (key > thr) | ((key == thr) & (gidx <= istar))
                row_v[sl] = jnp.where(keep, v, jnp.float32(0.0))

            descs_out[p] = pltpu.async_copy(row_v, out_hbm.at[base + i],
                                            sems_out[p])
        for d in descs_out:
            if d is not None:
                d.wait()

    sck = pl.kernel(
        body,
        out_type=jax.ShapeDtypeStruct((b, n), jnp.float32),
        mesh=mesh,
        scratch_types=[
            pltpu.VMEM((n,), jnp.float32),
            pltpu.VMEM((n,), jnp.float32),
            pltpu.VMEM((n,), jnp.int32),
            pltpu.VMEM((_NBKT * _L,), jnp.int32),
            pltpu.VMEM((_NBKT,), jnp.int32),
            pltpu.SemaphoreType.DMA,
            pltpu.SemaphoreType.DMA,
            pltpu.SemaphoreType.DMA,
            pltpu.SemaphoreType.DMA,
        ],
        compiler_params=pltpu.CompilerParams(needs_layout_passes=False),
        interpret=interpret,
    )

    return sck


_kern = _make_sc_kernel(128, 32768, 512)


def kernel(x):
    return _kern(x)


# speedup vs baseline: 1.1910x; 1.0878x over previous
import jax
import jax.numpy as jnp
from jax import lax
from jax.experimental import pallas as pl
from jax.experimental.pallas import tpu as pltpu, tpu_sc as plsc

_L = 16
_NBKT = 256


def _make_sc_kernel(b, n, k, nc=2, ns=16, interpret=False):
    nw = nc * ns
    rpw = b // nw
    nv = n // _L
    mesh = plsc.VectorSubcoreMesh(
        core_axis_name="c", subcore_axis_name="s",
        num_cores=nc, num_subcores=ns)

    def body(x_hbm, out_hbm, row_v, cand_v, out_v, hist_v, tot_v, kept_v,
             sem_out):
        cid = lax.axis_index("c")
        sid = lax.axis_index("s")
        wid = sid * nc + cid
        base = wid * rpw
        lanes = lax.broadcasted_iota(jnp.int32, (_L,), 0)
        laneoff = lanes * _NBKT
        zeros16 = jnp.zeros((_L,), jnp.int32)
        ones16 = jnp.ones((_L,), jnp.int32)
        fzeros16 = jnp.zeros((_L,), jnp.float32)
        kk = jnp.int32(k)

        def monotone_key(v):
            xi = plsc.bitcast(v, jnp.int32)
            return xi ^ ((xi >> 31) & jnp.int32(0x7FFFFFFF))

        @plsc.parallel_loop(0, nv, unroll=8)
        def _(j):
            out_v[pl.ds(j * _L, _L)] = fzeros16

        desc_out = None
        for i in range(rpw):
            pltpu.sync_copy(x_hbm.at[base + i], row_v)

            @plsc.parallel_loop(0, _NBKT, unroll=8)
            def _(j):
                hist_v[pl.ds(j * _L, _L)] = zeros16

            @plsc.parallel_loop(0, nv, unroll=8)
            def _(j):
                key = monotone_key(row_v[pl.ds(j * _L, _L)])
                bkt = (key >> 24) + 128
                plsc.addupdate_scatter(hist_v, [laneoff + bkt], ones16)

            def t_body(g, _):
                acc = zeros16
                for l in range(_L):
                    acc = acc + hist_v[pl.ds(l * _NBKT + g * _L, _L)]
                tot_v[pl.ds(g * _L, _L)] = acc
                return 0
            lax.fori_loop(0, _NBKT // _L, t_body, 0)

            def nb_body(t, carry):
                hi, cntb = carry
                g = jnp.int32(_NBKT // _L - 1) - t
                tg = tot_v[pl.ds(g * _L, _L)]
                suf = lax.rev(plsc.cumsum(lax.rev(tg, (0,))), (0,)) + hi
                cntb = cntb + jnp.sum((suf >= kk).astype(jnp.int32))
                return (hi + jnp.sum(tg), cntb)
            _, nbcnt = lax.fori_loop(0, _NBKT // _L, nb_body,
                                     (jnp.int32(0), jnp.int32(0)))
            b1 = nbcnt - 1

            def c_body(j, ptr):
                key = monotone_key(row_v[pl.ds(j * _L, _L)])
                bkt = (key >> 24) + 128
                m = bkt >= b1
                cs = plsc.cumsum(m.astype(jnp.int32))
                addr = ptr + cs - 1
                plsc.store_scatter(cand_v, [addr], j * _L + lanes, mask=m)
                return ptr + jnp.sum(m.astype(jnp.int32))
            ptr = plsc.parallel_loop(0, nv, carry=zeros16, unroll=8)(c_body)
            m1 = jnp.max(ptr)
            nv_c = (m1 + _L - 1) // _L

            key_top = (b1 - 128) << 24

            def gather_keys(j):
                valid = (j * _L + lanes) < m1
                idxv = cand_v[pl.ds(j * _L, _L)]
                xv = plsc.load_gather(row_v, [idxv], mask=valid)
                return monotone_key(xv), idxv, valid

            def bit_body(t, prefix):
                cand_t = prefix | (jnp.int32(1) << (jnp.int32(23) - t))

                def cnt_body(j, acc):
                    keyv, _, valid = gather_keys(j)
                    ge = (keyv >= cand_t) & valid
                    return acc + ge.astype(jnp.int32)
                accv = plsc.parallel_loop(0, nv_c, carry=zeros16,
                                          unroll=4)(cnt_body)
                cnt = jnp.sum(accv)
                return jnp.where(cnt >= kk, cand_t, prefix)
            thr = lax.fori_loop(0, 24, bit_body, key_top)

            def gt_body(j, acc):
                keyv, _, valid = gather_keys(j)
                gt = (keyv > thr) & valid
                return acc + gt.astype(jnp.int32)
            cgt = jnp.sum(plsc.parallel_loop(0, nv_c, carry=zeros16,
                                             unroll=4)(gt_body))
            r = kk - cgt

            def tie_body(j, carry):
                cnt, istar = carry
                keyv, idxv, valid = gather_keys(j)
                eq = (keyv == thr) & valid
                cs = plsc.cumsum(eq.astype(jnp.int32))
                hit = eq & ((cs + cnt) == r)
                istar = istar + jnp.sum(jnp.where(hit, idxv, 0))
                return (cnt + jnp.sum(eq.astype(jnp.int32)), istar)
            _, istar = lax.fori_loop(0, nv_c, tie_body,
                                     (jnp.int32(0), jnp.int32(0)))

            if i > 0:
                desc_out.wait()

                @plsc.parallel_loop(0, k // _L, unroll=4)
                def _(j):
                    idxv = kept_v[pl.ds(j * _L, _L)]
                    plsc.store_scatter(out_v, [idxv], fzeros16)

            def vs_body(j, wptr):
                keyv, idxv, valid = gather_keys(j)
                xv = plsc.load_gather(row_v, [idxv], mask=valid)
                keep = valid & ((keyv > thr)
                                | ((keyv == thr) & (idxv <= istar)))
                plsc.store_scatter(out_v, [idxv], xv, mask=keep)
                cs = plsc.cumsum(keep.astype(jnp.int32))
                plsc.store_scatter(kept_v, [wptr + cs - 1], idxv, mask=keep)
                return wptr + jnp.sum(keep.astype(jnp.int32))
            plsc.parallel_loop(0, nv_c, carry=zeros16, unroll=4)(vs_body)

            desc_out = pltpu.async_copy(out_v, out_hbm.at[base + i], sem_out)
        desc_out.wait()

    sck = pl.kernel(
        body,
        out_type=jax.ShapeDtypeStruct((b, n), jnp.float32),
        mesh=mesh,
        scratch_types=[
            pltpu.VMEM((n,), jnp.float32),
            pltpu.VMEM((n,), jnp.int32),
            pltpu.VMEM((n,), jnp.float32),
            pltpu.VMEM((_NBKT * _L,), jnp.int32),
            pltpu.VMEM((_NBKT,), jnp.int32),
            pltpu.VMEM((k,), jnp.int32),
            pltpu.SemaphoreType.DMA,
        ],
        compiler_params=pltpu.CompilerParams(needs_layout_passes=False),
        interpret=interpret,
    )

    return sck


_kern = _make_sc_kernel(128, 32768, 512)


def kernel(x):
    return _kern(x)
